# baseline (speedup 1.0000x reference)
import jax
import jax.numpy as jnp
from jax import lax
from jax.experimental import pallas as pl
from jax.experimental.pallas import tpu as pltpu
from jax.experimental.pallas import tpu_sc as plsc

_N = 512
_H = 128
_P = 128
_G = 4
_GG = _G * _G
_NW = 32
_APW = _N // _NW
_L = 16
_NC = _N // _L
_HC = _H // _L
_CAP = _N + 2 * _L
_TRASH = _CAP
_HB = _N // 2 + _L


def _pool_body(posx_hbm, posy_hbm, posx1_hbm, posy1_hbm, hid_hbm, out_hbm,
               posx_v, posy_v, posx1_v, posy1_v, hid_v, sl_v, sl2_v, offs_v,
               fl_v, acc_v):
    cid = lax.axis_index("c")
    sid = lax.axis_index("s")
    wid = sid * 2 + cid

    pltpu.sync_copy(posx_hbm, posx_v)
    pltpu.sync_copy(posy_hbm, posy_v)
    pltpu.sync_copy(posx1_hbm, posx1_v)
    pltpu.sync_copy(posy1_hbm, posy1_v)
    pltpu.sync_copy(hid_hbm, hid_v.at[pl.ds(0, _N)])

    for hc in range(_HC):
        hid_v[_N, pl.ds(hc * _L, _L)] = jnp.full((_L,), -3e38, jnp.float32)

    lanes = lax.iota(jnp.int32, _L)

    def per_agent(a, _):
        i = wid * _APW + a
        iv = jnp.full((_L,), i, jnp.int32)
        pxi = plsc.load_gather(posx1_v, [iv])
        pyi = plsc.load_gather(posy1_v, [iv])

        def chunk(jc, ms):
            m0, m1 = ms
            nm = []
            for h, m in ((0, m0), (1, m1)):
                jch = jc + h * (_NC // 2)
                px = posx_v[jch, :]
                py = posy_v[jch, :]
                relx = px - pxi
                rely = py - pyi
                inb = (jnp.abs(relx) <= 1.0) & (jnp.abs(rely) <= 1.0)
                jv = jch * _L + lanes
                valid = inb & (jv != i)
                gx = jnp.clip((relx + 1.0) * 2.0, 0.0,
                              _G - 1.0).astype(jnp.int32)
                gy = jnp.clip((rely + 1.0) * 2.0, 0.0,
                              _G - 1.0).astype(jnp.int32)
                val = (gx * _G + gy) * _N + jv
                vi = valid.astype(jnp.int32)
                offs = m + plsc.cumsum(vi) - 1
                plsc.store_scatter(sl_v, [offs], val, mask=valid)
                nm.append(offs[_L - 1] + 1)
            return tuple(nm)

        nv0, nv1 = lax.fori_loop(0, _NC // 2, chunk,
                                 (jnp.int32(0), jnp.int32(_HB)))
        nv1 = nv1 - _HB
        nkc0 = (nv0 + _L - 1) >> 4
        nkc1 = (nv1 + _L - 1) >> 4
        nkcm = jnp.maximum(nkc0, nkc1)

        plsc.store_scatter(sl_v, [nv0 + lanes],
                           jnp.full((_L,), _GG * _N, jnp.int32))
        plsc.store_scatter(sl_v, [_HB + nv1 + lanes],
                           jnp.full((_L,), _GG * _N, jnp.int32))

        offs_v[pl.ds(0, _L)] = jnp.zeros((_L,), jnp.int32)
        offs_v[pl.ds(_L, _L)] = jnp.full((_L,), _TRASH, jnp.int32)
        offs_v[pl.ds(2 * _L, _L)] = jnp.zeros((_L,), jnp.int32)
        offs_v[pl.ds(3 * _L, _L)] = jnp.full((_L,), _TRASH + 4 * _L,
                                             jnp.int32)

        def cchunk(kc, _):
            for base, tb, nk in ((0, 0, nkc0), (_HB, 2 * _L, nkc1)):
                valv = sl_v[pl.ds(base + kc * _L, _L)]
                cv = ((valv >> 9) & 31) + tb
                occ, lastm = plsc.scan_count(cv)
                bs = plsc.load_gather(offs_v, [cv])
                inr = jnp.full((_L,), kc < nk)
                plsc.store_scatter(offs_v, [cv], bs + occ,
                                   mask=lastm & inr)
            return 0

        lax.fori_loop(0, nkcm, cchunk, 0)

        cnta = offs_v[pl.ds(0, _L)]
        cntv = cnta + offs_v[pl.ds(2 * _L, _L)]
        cs = plsc.cumsum(cntv)
        starts = cs - cntv
        offs_v[pl.ds(0, _L)] = starts
        offs_v[pl.ds(_L, _L)] = jnp.full((_L,), _TRASH, jnp.int32)
        offs_v[pl.ds(2 * _L, _L)] = starts + cnta
        offs_v[pl.ds(3 * _L, _L)] = jnp.full((_L,), _TRASH + 4 * _L,
                                             jnp.int32)

        nv = nv0 + nv1
        nkc = (nv + _L - 1) >> 4

        def flinit(kc, _):
            fl_v[pl.ds(kc * _L, _L)] = jnp.full((_L,), _GG, jnp.int32)
            return 0

        lax.fori_loop(0, nkc, flinit, 0)
        plsc.store_scatter(fl_v, [cs - 1], lanes, mask=cntv > 0)

        def schunk(kc, _):
            for base, tb, nk in ((0, 0, nkc0), (_HB, 2 * _L, nkc1)):
                valv = sl_v[pl.ds(base + kc * _L, _L)]
                cv = ((valv >> 9) & 31) + tb
                jv = valv & (_N - 1)
                occ, lastm = plsc.scan_count(cv)
                bs = plsc.load_gather(offs_v, [cv])
                inr = jnp.full((_L,), kc < nk)
                plsc.store_scatter(sl2_v, [bs + occ - 1], jv, mask=inr)
                plsc.store_scatter(offs_v, [cv], bs + occ,
                                   mask=lastm & inr)
            return 0

        lax.fori_loop(0, nkcm, schunk, 0)

        plsc.store_scatter(sl2_v, [nv + lanes], jnp.full((_L,), _N, jnp.int32))

        p = a & 3

        def initr(r, _):
            for hc in range(_HC):
                acc_v[p * _GG + r, pl.ds(hc * _L, _L)] = jnp.zeros(
                    (_L,), jnp.float32)
            return 0

        lax.fori_loop(0, _GG, initr, 0)

        neg = jnp.full((_L,), -3e38, jnp.float32)

        def walk(kc, regs):
            jv16 = sl2_v[pl.ds(kc * _L, _L)]
            flv = fl_v[pl.ds(kc * _L, _L)]
            nregs = list(regs)
            for l in range(_L):
                j = jv16[l]
                for hc in range(_HC):
                    nregs[hc] = jnp.maximum(
                        nregs[hc], hid_v[j, pl.ds(hc * _L, _L)])
                fc = flv[l]
                row = jnp.where(fc < _GG, p * _GG + fc, 4 * _GG)
                for hc in range(_HC):
                    acc_v[row, pl.ds(hc * _L, _L)] = nregs[hc]
                fm = jnp.full((_L,), fc < _GG)
                for hc in range(_HC):
                    nregs[hc] = jnp.where(fm, neg, nregs[hc])
            return tuple(nregs)

        init = tuple(neg for _ in range(_HC))
        lax.fori_loop(0, nkc, walk, init)

        @pl.when(p == 3)
        def _():
            pltpu.sync_copy(acc_v.at[pl.ds(0, 4 * _GG)],
                            out_hbm.at[pl.ds((i - 3) * _GG, 4 * _GG)])

        return 0

    lax.fori_loop(0, _APW, per_agent, 0)


def _mm_body(g_ref, w_ref, b_ref, o_ref):
    o_ref[...] = jnp.maximum(
        jnp.dot(g_ref[...], w_ref[...], preferred_element_type=jnp.float32)
        + b_ref[...],
        0.0,
    )


def kernel(pos, hidden, W, b):
    posx = pos[:, 0].reshape(_NC, _L)
    posy = pos[:, 1].reshape(_NC, _L)

    grid = pl.kernel(
        _pool_body,
        out_type=jax.ShapeDtypeStruct((_N * _GG, _H), jnp.float32),
        mesh=plsc.VectorSubcoreMesh(core_axis_name="c", subcore_axis_name="s"),
        scratch_types=[
            pltpu.VMEM((_NC, _L), jnp.float32),
            pltpu.VMEM((_NC, _L), jnp.float32),
            pltpu.VMEM((_N,), jnp.float32),
            pltpu.VMEM((_N,), jnp.float32),
            pltpu.VMEM((_N + 1, _H), jnp.float32),
            pltpu.VMEM((2 * _HB,), jnp.int32),
            pltpu.VMEM((_CAP + _N + _L,), jnp.int32),
            pltpu.VMEM((4 * _L,), jnp.int32),
            pltpu.VMEM((_CAP,), jnp.int32),
            pltpu.VMEM((4 * _GG + 1, _H), jnp.float32),
        ],
        compiler_params=pltpu.CompilerParams(needs_layout_passes=False),
    )(posx, posy, posx.reshape(_N), posy.reshape(_N), hidden)

    return pl.pallas_call(
        _mm_body,
        out_shape=jax.ShapeDtypeStruct((_N, _P), jnp.float32),
    )(grid.reshape(_N, _GG * _H), W, b.reshape(1, _P))


# speedup vs baseline: 1.0212x; 1.0212x over previous
import jax
import jax.numpy as jnp
from jax import lax
from jax.experimental import pallas as pl
from jax.experimental.pallas import tpu as pltpu
from jax.experimental.pallas import tpu_sc as plsc

_N = 512
_H = 128
_P = 128
_G = 4
_GG = _G * _G
_NW = 32
_APW = _N // _NW
_L = 16
_NC = _N // _L
_HC = _H // _L
_CAP = _N + 2 * _L
_TRASH = _CAP


def _pool_body(posx_hbm, posy_hbm, posx1_hbm, posy1_hbm, hid_hbm, out_hbm,
               posx_v, posy_v, posx1_v, posy1_v, hid_v, sl_v, sl2_v, offs_v,
               fl_v, acc_v):
    cid = lax.axis_index("c")
    sid = lax.axis_index("s")
    wid = sid * 2 + cid

    pltpu.sync_copy(posx_hbm, posx_v)
    pltpu.sync_copy(posy_hbm, posy_v)
    pltpu.sync_copy(posx1_hbm, posx1_v)
    pltpu.sync_copy(posy1_hbm, posy1_v)
    pltpu.sync_copy(hid_hbm, hid_v.at[pl.ds(0, _N)])

    for hc in range(_HC):
        hid_v[_N, pl.ds(hc * _L, _L)] = jnp.full((_L,), -3e38, jnp.float32)

    lanes = lax.iota(jnp.int32, _L)

    def per_agent(a, _):
        i = wid * _APW + a
        iv = jnp.full((_L,), i, jnp.int32)
        pxi = plsc.load_gather(posx1_v, [iv])
        pyi = plsc.load_gather(posy1_v, [iv])

        def chunk(jc, m):
            px = posx_v[jc, :]
            py = posy_v[jc, :]
            relx = px - pxi
            rely = py - pyi
            inb = (jnp.abs(relx) <= 1.0) & (jnp.abs(rely) <= 1.0)
            jv = jc * _L + lanes
            valid = inb & (jv != i)
            gx = jnp.clip((relx + 1.0) * 2.0, 0.0, _G - 1.0).astype(jnp.int32)
            gy = jnp.clip((rely + 1.0) * 2.0, 0.0, _G - 1.0).astype(jnp.int32)
            val = (gx * _G + gy) * _N + jv
            vi = valid.astype(jnp.int32)
            offs = m + plsc.cumsum(vi) - 1
            plsc.store_scatter(sl_v, [offs], val, mask=valid)
            return offs[_L - 1] + 1

        nv = lax.fori_loop(0, _NC, chunk, jnp.int32(0))
        nkc = (nv + _L - 1) >> 4

        plsc.store_scatter(sl_v, [nv + lanes],
                           jnp.full((_L,), _GG * _N, jnp.int32))

        offs_v[pl.ds(0, _L)] = jnp.zeros((_L,), jnp.int32)
        offs_v[pl.ds(_L, _L)] = jnp.full((_L,), _TRASH, jnp.int32)

        def cchunk(kc, _):
            valv = sl_v[pl.ds(kc * _L, _L)]
            cv = valv >> 9
            occ, lastm = plsc.scan_count(cv)
            base = plsc.load_gather(offs_v, [cv])
            plsc.store_scatter(offs_v, [cv], base + occ, mask=lastm)
            return 0

        lax.fori_loop(0, nkc, cchunk, 0)

        cntv = offs_v[pl.ds(0, _L)]
        cs = plsc.cumsum(cntv)
        offs_v[pl.ds(0, _L)] = cs - cntv
        offs_v[pl.ds(_L, _L)] = jnp.full((_L,), _TRASH, jnp.int32)

        def flinit(kc, _):
            fl_v[pl.ds(kc * _L, _L)] = jnp.full((_L,), _GG, jnp.int32)
            return 0

        lax.fori_loop(0, nkc, flinit, 0)
        plsc.store_scatter(fl_v, [cs - 1], lanes, mask=cntv > 0)

        def schunk(kc, _):
            valv = sl_v[pl.ds(kc * _L, _L)]
            cv = valv >> 9
            jv = valv & (_N - 1)
            occ, lastm = plsc.scan_count(cv)
            base = plsc.load_gather(offs_v, [cv])
            plsc.store_scatter(sl2_v, [base + occ - 1], jv)
            plsc.store_scatter(offs_v, [cv], base + occ, mask=lastm)
            return 0

        lax.fori_loop(0, nkc, schunk, 0)

        plsc.store_scatter(sl2_v, [nv + lanes], jnp.full((_L,), _N, jnp.int32))

        p = a & 3

        def initr(r, _):
            for hc in range(_HC):
                acc_v[p * _GG + r, pl.ds(hc * _L, _L)] = jnp.zeros(
                    (_L,), jnp.float32)
            return 0

        lax.fori_loop(0, _GG, initr, 0)

        neg = jnp.full((_L,), -3e38, jnp.float32)

        def walk(kc, regs):
            jv16 = sl2_v[pl.ds(kc * _L, _L)]
            flv = fl_v[pl.ds(kc * _L, _L)]
            nregs = list(regs)
            for l in range(_L):
                j = jv16[l]
                for hc in range(_HC):
                    nregs[hc] = jnp.maximum(
                        nregs[hc], hid_v[j, pl.ds(hc * _L, _L)])
                fc = flv[l]
                row = jnp.where(fc < _GG, p * _GG + fc, 4 * _GG)
                for hc in range(_HC):
                    acc_v[row, pl.ds(hc * _L, _L)] = nregs[hc]
                fm = jnp.full((_L,), fc < _GG)
                for hc in range(_HC):
                    nregs[hc] = jnp.where(fm, neg, nregs[hc])
            return tuple(nregs)

        init = tuple(neg for _ in range(_HC))
        lax.fori_loop(0, nkc, walk, init)

        @pl.when(p == 3)
        def _():
            pltpu.sync_copy(acc_v.at[pl.ds(0, 4 * _GG)],
                            out_hbm.at[pl.ds((i - 3) * _GG, 4 * _GG)])

        return 0

    lax.fori_loop(0, _APW, per_agent, 0)


def _mm_body(g_ref, w_ref, b_ref, o_ref):
    o_ref[...] = jnp.maximum(
        jnp.dot(g_ref[...], w_ref[...], preferred_element_type=jnp.float32)
        + b_ref[...],
        0.0,
    )


def kernel(pos, hidden, W, b):
    posx = pos[:, 0].reshape(_NC, _L)
    posy = pos[:, 1].reshape(_NC, _L)

    grid = pl.kernel(
        _pool_body,
        out_type=jax.ShapeDtypeStruct((_N * _GG, _H), jnp.float32),
        mesh=plsc.VectorSubcoreMesh(core_axis_name="c", subcore_axis_name="s"),
        scratch_types=[
            pltpu.VMEM((_NC, _L), jnp.float32),
            pltpu.VMEM((_NC, _L), jnp.float32),
            pltpu.VMEM((_N,), jnp.float32),
            pltpu.VMEM((_N,), jnp.float32),
            pltpu.VMEM((_N + 1, _H), jnp.float32),
            pltpu.VMEM((_N + _L,), jnp.int32),
            pltpu.VMEM((_CAP + _N + _L,), jnp.int32),
            pltpu.VMEM((2 * _L,), jnp.int32),
            pltpu.VMEM((_CAP,), jnp.int32),
            pltpu.VMEM((4 * _GG + 1, _H), jnp.float32),
        ],
        compiler_params=pltpu.CompilerParams(needs_layout_passes=False),
    )(posx, posy, posx.reshape(_N), posy.reshape(_N), hidden)

    return pl.pallas_call(
        _mm_body,
        out_shape=jax.ShapeDtypeStruct((_N, _P), jnp.float32),
    )(grid.reshape(_N, _GG * _H), W, b.reshape(1, _P))
